# baseline (speedup 1.0000x reference)
import jax
import jax.numpy as jnp
from jax.experimental import pallas as pl
from jax.experimental.pallas import tpu as pltpu

_TEMPERATURE = 64.0
_N_CHUNKS = 4


def _protonet_body(s_ref, q_hbm, o_ref, q_vmem, *sems):
    n_q = q_vmem.shape[0]
    chunk = n_q // _N_CHUNKS
    cps = []
    for c in range(_N_CHUNKS):
        cp = pltpu.make_async_copy(
            q_hbm.at[pl.ds(c * chunk, chunk)],
            q_vmem.at[pl.ds(c * chunk, chunk)],
            sems[c],
        )
        cp.start()
        cps.append(cp)

    proto = jnp.sum(s_ref[...], axis=0) * (1.0 / s_ref.shape[0])
    pn = jnp.sum(proto * proto, axis=1)[None, :]

    for c in range(_N_CHUNKS):
        cps[c].wait()
        q = q_vmem[pl.ds(c * chunk, chunk), :]
        qn = jnp.sum(q * q, axis=1, keepdims=True)
        cross = jax.lax.dot_general(
            q, proto, (((1,), (1,)), ((), ())),
            preferred_element_type=jnp.float32,
        )
        o_ref[pl.ds(c * chunk, chunk), :] = (
            (2.0 * cross - qn - pn) * (1.0 / _TEMPERATURE))


def kernel(support, query):
    n_batch, n_shot, n_way, emb_dim = support.shape
    n_query = n_batch * query.shape[1] * n_way
    s = support.reshape(n_shot, n_way, emb_dim)
    q = query.reshape(n_query, emb_dim)
    return pl.pallas_call(
        _protonet_body,
        in_specs=[
            pl.BlockSpec((n_shot, n_way, emb_dim), lambda: (0, 0, 0)),
            pl.BlockSpec(memory_space=pl.ANY),
        ],
        out_shape=jax.ShapeDtypeStruct((n_query, n_way), jnp.float32),
        scratch_shapes=[pltpu.VMEM((n_query, emb_dim), jnp.float32)]
        + [pltpu.SemaphoreType.DMA] * _N_CHUNKS,
    )(s, q)


# speedup vs baseline: 1.2339x; 1.2339x over previous
import jax
import jax.numpy as jnp
from jax.experimental import pallas as pl

_TEMPERATURE = 64.0


def _protonet_body(s_ref, q_ref, o_ref):
    inv_t = 1.0 / _TEMPERATURE
    proto = jnp.sum(s_ref[...], axis=0) * (1.0 / s_ref.shape[0])
    q = q_ref[...]
    qn = jnp.sum(q * q, axis=1, keepdims=True) * inv_t
    pn = (jnp.sum(proto * proto, axis=1) * inv_t)[None, :]
    cross = jax.lax.dot_general(
        q, proto * (2.0 * inv_t), (((1,), (1,)), ((), ())),
        preferred_element_type=jnp.float32,
    )
    o_ref[...] = cross - qn - pn


def kernel(support, query):
    n_batch, n_shot, n_way, emb_dim = support.shape
    n_query = n_batch * query.shape[1] * n_way
    s = support.reshape(n_shot, n_way, emb_dim)
    q = query.reshape(n_query, emb_dim)
    return pl.pallas_call(
        _protonet_body,
        out_shape=jax.ShapeDtypeStruct((n_query, n_way), jnp.float32),
    )(s, q)
